# baseline (speedup 1.0000x reference)
import jax
import jax.numpy as jnp
from jax import lax
from jax.experimental import pallas as pl
from jax.experimental.pallas import tpu as pltpu
from jax.experimental.pallas import tpu_sc as plsc

_N = 50000
_H = 256
_B = 512
_T = 2

_NB = 2000
_G = _N // _NB

_C = 256
_NW = 32
_SPW = _B // _NW
_RMAX = _N - _C

_mesh = plsc.VectorSubcoreMesh(
    core_axis_name="c", subcore_axis_name="s", num_cores=2, num_subcores=16)


def _full(shape):
    return pl.BlockSpec(shape, lambda *a: tuple(0 for _ in shape))


def _onehot_t(batch_ref):
    b2 = batch_ref[0]
    iota = jax.lax.broadcasted_iota(jnp.int32, (_B, _NB), 0)
    return iota == b2


def _k1_body(node_ref, w_ref, batch_ref, att_ref, counts_ref):
    i = pl.program_id(0)
    att_ref[...] = jax.lax.dot_general(
        node_ref[...], w_ref[...], (((1,), (1,)), ((), ())),
        preferred_element_type=jnp.float32)
    cnt = jnp.sum(_onehot_t(batch_ref).astype(jnp.float32), axis=1,
                  keepdims=True)

    @pl.when(i == 0)
    def _():
        counts_ref[...] = jnp.zeros_like(counts_ref)

    counts_ref[...] += cnt


def _ka_body(att_ref, gf_ref, batch_ref, scores_ref, m_ref):
    i = pl.program_id(0)
    st = jax.lax.dot_general(gf_ref[...], att_ref[...],
                             (((1,), (1,)), ((), ())),
                             preferred_element_type=jnp.float32)
    ptb = _onehot_t(batch_ref)
    scores_ref[0, 0, :] = jnp.sum(jnp.where(ptb, st, 0.0), axis=0)
    mt = jnp.max(jnp.where(ptb, st, -jnp.inf), axis=1, keepdims=True)

    @pl.when(i == 0)
    def _():
        m_ref[...] = jnp.full_like(m_ref, -jnp.inf)

    m_ref[...] = jnp.maximum(m_ref[...], mt)


def _ke_body(batch_ref, scores_ref, m_ref, e_ref, ssum_ref):
    i = pl.program_id(0)
    ptb = _onehot_t(batch_ref)
    m = m_ref[...]
    mfix = jnp.where(jnp.isfinite(m), m, 0.0)
    msel = jnp.sum(jnp.where(ptb, mfix, 0.0), axis=0)
    e = jnp.exp(scores_ref[0, 0, :] - msel)
    e_ref[0, 0, :] = e

    @pl.when(i == 0)
    def _():
        ssum_ref[...] = jnp.zeros_like(ssum_ref)

    ssum_ref[...] += jnp.sum(jnp.where(ptb, e[None, :], 0.0), axis=1,
                             keepdims=True)


def _kst_body(counts_ref, st_ref):
    ir = jax.lax.broadcasted_iota(jnp.int32, (_B, _B), 0)
    ic = jax.lax.broadcasted_iota(jnp.int32, (_B, _B), 1)
    tril = (ic < ir).astype(jnp.float32)
    st = jax.lax.dot_general(tril, counts_ref[...], (((1,), (0,)), ((), ())),
                             preferred_element_type=jnp.float32)
    st_ref[...] = st.astype(jnp.int32)


def _k2_body(sums_ref, counts_ref, gf_ref):
    gf_ref[...] = sums_ref[...] / jnp.maximum(counts_ref[...], 1.0)


def _kg_body(csum_ref, ssum_ref, gf_ref, wih_ref, whh_ref, bih_ref,
             bhh_ref, out_ref):
    ctx = csum_ref[...] / (ssum_ref[...] + 1e-16)
    h = gf_ref[...]
    gi = jax.lax.dot_general(ctx, wih_ref[...], (((1,), (1,)), ((), ())),
                             preferred_element_type=jnp.float32)
    gi = gi + bih_ref[...][None, :]
    gh = jax.lax.dot_general(h, whh_ref[...], (((1,), (1,)), ((), ())),
                             preferred_element_type=jnp.float32)
    gh = gh + bhh_ref[...][None, :]
    r = jax.nn.sigmoid(gi[:, 0:_H] + gh[:, 0:_H])
    z = jax.nn.sigmoid(gi[:, _H:2 * _H] + gh[:, _H:2 * _H])
    n = jnp.tanh(gi[:, 2 * _H:] + r * gh[:, 2 * _H:])
    out_ref[...] = (1.0 - z) * n + z * h


def _seg_body(weighted, node_hbm, batch_hbm, starts_hbm, e_hbm,
              out_hbm, rows, idx_v, sc_v, st_v, acc):
    cid = lax.axis_index("c")
    sid = lax.axis_index("s")
    wid = sid * 2 + cid
    b0 = wid * _SPW

    pltpu.sync_copy(starts_hbm, st_v.at[pl.ds(0, _B)])
    st_v[pl.ds(_B, 16)] = jnp.full((16,), _N, jnp.int32)

    sblk = st_v[pl.ds(b0, 16)]
    r_lo = sblk[0]
    r_hi = st_v[pl.ds(b0 + _SPW, 16)][0]

    z16 = jnp.zeros((16,), jnp.float32)

    def zrow(r, carry):
        def zcol(k, carry2):
            acc[r, pl.ds(k * 16, 16)] = z16
            return carry2
        return lax.fori_loop(0, _H // 16, zcol, carry)
    lax.fori_loop(0, _SPW, zrow, 0)

    a_lo = (r_lo // 8) * 8
    nchunks = (r_hi - a_lo + _C - 1) // _C

    def chunk(j, carry):
        r0 = a_lo + j * _C
        r0c = jnp.minimum(r0, _RMAX)
        r0c = pl.multiple_of(r0c, 8)
        pltpu.sync_copy(batch_hbm.at[pl.ds(r0c, _C)], idx_v)
        pltpu.sync_copy(node_hbm.at[pl.ds(r0c, _C), :], rows)
        if weighted:
            pltpu.sync_copy(e_hbm.at[pl.ds(r0c, _C)], sc_v)

        def group(g, carry2):
            off = g * 16
            seg = idx_v[pl.ds(off, 16)]
            loc = seg - b0
            gidx = r0c + off + lax.iota(jnp.int32, 16)
            valid = ((loc >= 0) & (loc < _SPW) & (gidx >= r0)
                     & (gidx < r_hi))
            if weighted:
                coef = jnp.where(valid, sc_v[pl.ds(off, 16)], 0.0)
            else:
                coef = jnp.where(valid, 1.0, 0.0)
            cvecs = [jnp.full((16,), coef[rr], jnp.float32)
                     for rr in range(16)]
            s0 = seg[0]
            g0 = r0c + off
            onseg = ((s0 == seg[15]) & (s0 >= b0) & (s0 < b0 + _SPW)
                     & (g0 >= r0) & (g0 + 15 < r_hi))

            @pl.when(onseg)
            def _fast():
                l = s0 - b0
                for k in range(_H // 16):
                    sl = pl.ds(k * 16, 16)
                    t = rows[off, sl] * cvecs[0]
                    for rr in range(1, 16):
                        t = t + rows[off + rr, sl] * cvecs[rr]
                    acc[l, sl] = acc[l, sl] + t

            @pl.when(jnp.logical_not(onseg))
            def _slow():
                locc = jnp.clip(loc, 0, _SPW - 1)
                for rr in range(16):
                    r = off + rr
                    l = locc[rr]
                    for k in range(_H // 16):
                        sl = pl.ds(k * 16, 16)
                        acc[l, sl] = acc[l, sl] + rows[r, sl] * cvecs[rr]
            return carry2
        lax.fori_loop(0, _C // 16, group, carry)
        return carry
    lax.fori_loop(0, nchunks, chunk, 0)

    base = pl.multiple_of(b0, _SPW)
    pltpu.sync_copy(acc, out_hbm.at[pl.ds(base, _SPW)])


def _s1_body(node_hbm, batch_hbm, starts_hbm, out_hbm,
             rows, idx_v, st_v, acc):
    _seg_body(False, node_hbm, batch_hbm, starts_hbm, None,
              out_hbm, rows, idx_v, None, st_v, acc)


def _c_body(node_hbm, batch_hbm, starts_hbm, e_hbm, out_hbm,
            rows, idx_v, sc_v, st_v, acc):
    _seg_body(True, node_hbm, batch_hbm, starts_hbm, e_hbm,
              out_hbm, rows, idx_v, sc_v, st_v, acc)


_out_type = jax.ShapeDtypeStruct((_B, _H), jnp.float32)

_s1 = pl.kernel(
    _s1_body, out_type=_out_type, mesh=_mesh,
    scratch_types=[
        pltpu.VMEM((_C, _H), jnp.float32),
        pltpu.VMEM((_C,), jnp.int32),
        pltpu.VMEM((_B + 16,), jnp.int32),
        pltpu.VMEM((_SPW, _H), jnp.float32),
    ])

_c = pl.kernel(
    _c_body, out_type=_out_type, mesh=_mesh,
    scratch_types=[
        pltpu.VMEM((_C, _H), jnp.float32),
        pltpu.VMEM((_C,), jnp.int32),
        pltpu.VMEM((_C,), jnp.float32),
        pltpu.VMEM((_B + 16,), jnp.int32),
        pltpu.VMEM((_SPW, _H), jnp.float32),
    ])


def kernel(node_feats, batch, W_attend, gru_w_ih, gru_w_hh, gru_b_ih,
           gru_b_hh):
    batch3 = batch.reshape(_G, 1, _NB)
    node_spec = pl.BlockSpec((_NB, _H), lambda i: (i, 0))
    batch_spec = pl.BlockSpec((1, 1, _NB), lambda i: (i, 0, 0))
    scores_spec = pl.BlockSpec((1, 1, _NB), lambda i: (i, 0, 0))

    att, counts = pl.pallas_call(
        _k1_body,
        grid=(_G,),
        in_specs=[node_spec, _full((_H, _H)), batch_spec],
        out_specs=[node_spec, _full((_B, 1))],
        out_shape=[
            jax.ShapeDtypeStruct((_N, _H), jnp.float32),
            jax.ShapeDtypeStruct((_B, 1), jnp.float32),
        ],
    )(node_feats, W_attend, batch3)

    starts = pl.pallas_call(
        _kst_body,
        in_specs=[_full((_B, 1))],
        out_specs=_full((_B, 1)),
        out_shape=jax.ShapeDtypeStruct((_B, 1), jnp.int32),
    )(counts)
    starts1 = starts.reshape(_B)
    sums = _s1(node_feats, batch, starts1)

    gf = pl.pallas_call(
        _k2_body,
        in_specs=[_full((_B, _H)), _full((_B, 1))],
        out_specs=_full((_B, _H)),
        out_shape=jax.ShapeDtypeStruct((_B, _H), jnp.float32),
    )(sums, counts)

    for _ in range(_T):
        scores, m = pl.pallas_call(
            _ka_body,
            grid=(_G,),
            in_specs=[node_spec, _full((_B, _H)), batch_spec],
            out_specs=[scores_spec, _full((_B, 1))],
            out_shape=[
                jax.ShapeDtypeStruct((_G, 1, _NB), jnp.float32),
                jax.ShapeDtypeStruct((_B, 1), jnp.float32),
            ],
        )(att, gf, batch3)

        ev, ssum = pl.pallas_call(
            _ke_body,
            grid=(_G,),
            in_specs=[batch_spec, scores_spec, _full((_B, 1))],
            out_specs=[scores_spec, _full((_B, 1))],
            out_shape=[
                jax.ShapeDtypeStruct((_G, 1, _NB), jnp.float32),
                jax.ShapeDtypeStruct((_B, 1), jnp.float32),
            ],
        )(batch3, scores, m)

        csum = _c(node_feats, batch, starts1, ev.reshape(_N))

        gf = pl.pallas_call(
            _kg_body,
            in_specs=[_full((_B, _H)), _full((_B, 1)), _full((_B, _H)),
                      _full((3 * _H, _H)), _full((3 * _H, _H)),
                      _full((3 * _H,)), _full((3 * _H,))],
            out_specs=_full((_B, _H)),
            out_shape=jax.ShapeDtypeStruct((_B, _H), jnp.float32),
        )(csum, ssum, gf, gru_w_ih, gru_w_hh, gru_b_ih, gru_b_hh)

    return gf


# speedup vs baseline: 1.1121x; 1.1121x over previous
import jax
import jax.numpy as jnp
from jax import lax
from jax.experimental import pallas as pl
from jax.experimental.pallas import tpu as pltpu
from jax.experimental.pallas import tpu_sc as plsc

_N = 50000
_H = 256
_B = 512
_T = 2

_NB = 2000
_G = _N // _NB

_C = 128
_NW = 32
_SPW = _B // _NW
_RMAX = _N - _C

_mesh = plsc.VectorSubcoreMesh(
    core_axis_name="c", subcore_axis_name="s", num_cores=2, num_subcores=16)


def _full(shape):
    return pl.BlockSpec(shape, lambda *a: tuple(0 for _ in shape))


def _onehot_t(batch_ref):
    b2 = batch_ref[0]
    iota = jax.lax.broadcasted_iota(jnp.int32, (_B, _NB), 0)
    return iota == b2


def _k1_body(node_ref, w_ref, batch_ref, att_ref, counts_ref):
    i = pl.program_id(0)
    att_ref[...] = jax.lax.dot_general(
        node_ref[...], w_ref[...], (((1,), (1,)), ((), ())),
        preferred_element_type=jnp.float32)
    cnt = jnp.sum(_onehot_t(batch_ref).astype(jnp.float32), axis=1,
                  keepdims=True)

    @pl.when(i == 0)
    def _():
        counts_ref[...] = jnp.zeros_like(counts_ref)

    counts_ref[...] += cnt


def _ka_body(att_ref, gf_ref, batch_ref, scores_ref, m_ref):
    i = pl.program_id(0)
    st = jax.lax.dot_general(gf_ref[...], att_ref[...],
                             (((1,), (1,)), ((), ())),
                             preferred_element_type=jnp.float32)
    ptb = _onehot_t(batch_ref)
    scores_ref[0, 0, :] = jnp.sum(jnp.where(ptb, st, 0.0), axis=0)
    mt = jnp.max(jnp.where(ptb, st, -jnp.inf), axis=1, keepdims=True)

    @pl.when(i == 0)
    def _():
        m_ref[...] = jnp.full_like(m_ref, -jnp.inf)

    m_ref[...] = jnp.maximum(m_ref[...], mt)


def _ke_body(batch_ref, scores_ref, m_ref, e_ref, ssum_ref):
    i = pl.program_id(0)
    ptb = _onehot_t(batch_ref)
    m = m_ref[...]
    mfix = jnp.where(jnp.isfinite(m), m, 0.0)
    msel = jnp.sum(jnp.where(ptb, mfix, 0.0), axis=0)
    e = jnp.exp(scores_ref[0, 0, :] - msel)
    e_ref[0, 0, :] = e

    @pl.when(i == 0)
    def _():
        ssum_ref[...] = jnp.zeros_like(ssum_ref)

    ssum_ref[...] += jnp.sum(jnp.where(ptb, e[None, :], 0.0), axis=1,
                             keepdims=True)


def _kst_body(counts_ref, st_ref):
    ir = jax.lax.broadcasted_iota(jnp.int32, (_B, _B), 0)
    ic = jax.lax.broadcasted_iota(jnp.int32, (_B, _B), 1)
    tril = (ic < ir).astype(jnp.float32)
    st = jax.lax.dot_general(tril, counts_ref[...], (((1,), (0,)), ((), ())),
                             preferred_element_type=jnp.float32)
    st_ref[...] = st.astype(jnp.int32)


def _k2_body(sums_ref, counts_ref, gf_ref):
    gf_ref[...] = sums_ref[...] / jnp.maximum(counts_ref[...], 1.0)


def _kg_body(csum_ref, ssum_ref, gf_ref, wih_ref, whh_ref, bih_ref,
             bhh_ref, out_ref):
    ctx = csum_ref[...] / (ssum_ref[...] + 1e-16)
    h = gf_ref[...]
    gi = jax.lax.dot_general(ctx, wih_ref[...], (((1,), (1,)), ((), ())),
                             preferred_element_type=jnp.float32)
    gi = gi + bih_ref[...][None, :]
    gh = jax.lax.dot_general(h, whh_ref[...], (((1,), (1,)), ((), ())),
                             preferred_element_type=jnp.float32)
    gh = gh + bhh_ref[...][None, :]
    r = jax.nn.sigmoid(gi[:, 0:_H] + gh[:, 0:_H])
    z = jax.nn.sigmoid(gi[:, _H:2 * _H] + gh[:, _H:2 * _H])
    n = jnp.tanh(gi[:, 2 * _H:] + r * gh[:, 2 * _H:])
    out_ref[...] = (1.0 - z) * n + z * h


def _seg_body(weighted, node_hbm, batch_hbm, starts_hbm, e_hbm,
              out_hbm, rows, idx_v, sc_v, st_v, acc):
    cid = lax.axis_index("c")
    sid = lax.axis_index("s")
    wid = sid * 2 + cid
    b0 = wid * _SPW

    pltpu.sync_copy(starts_hbm, st_v.at[pl.ds(0, _B)])
    st_v[pl.ds(_B, 16)] = jnp.full((16,), _N, jnp.int32)

    sblk = st_v[pl.ds(b0, 16)]
    r_lo = sblk[0]
    r_hi = st_v[pl.ds(b0 + _SPW, 16)][0]

    z16 = jnp.zeros((16,), jnp.float32)

    def zrow(r, carry):
        def zcol(k, carry2):
            acc[r, pl.ds(k * 16, 16)] = z16
            return carry2
        return lax.fori_loop(0, _H // 16, zcol, carry)
    lax.fori_loop(0, _SPW, zrow, 0)

    a_lo = (r_lo // 8) * 8
    nchunks = (r_hi - a_lo + _C - 1) // _C

    def chunk(j, carry):
        r0 = a_lo + j * _C
        r0c = jnp.minimum(r0, _RMAX)
        r0c = pl.multiple_of(r0c, 8)
        pltpu.sync_copy(batch_hbm.at[pl.ds(r0c, _C)], idx_v)
        pltpu.sync_copy(node_hbm.at[pl.ds(r0c, _C), :], rows)
        if weighted:
            pltpu.sync_copy(e_hbm.at[pl.ds(r0c, _C)], sc_v)

        def group(g, carry2):
            off = g * 16
            seg = idx_v[pl.ds(off, 16)]
            loc = seg - b0
            gidx = r0c + off + lax.iota(jnp.int32, 16)
            valid = ((loc >= 0) & (loc < _SPW) & (gidx >= r0)
                     & (gidx < r_hi))
            if weighted:
                coef = jnp.where(valid, sc_v[pl.ds(off, 16)], 0.0)
            else:
                coef = jnp.where(valid, 1.0, 0.0)
            cvecs = [jnp.full((16,), coef[rr], jnp.float32)
                     for rr in range(16)]
            s0 = seg[0]
            g0 = r0c + off
            onseg = ((s0 == seg[15]) & (s0 >= b0) & (s0 < b0 + _SPW)
                     & (g0 >= r0) & (g0 + 15 < r_hi))

            @pl.when(onseg)
            def _fast():
                l = s0 - b0
                for k in range(_H // 16):
                    sl = pl.ds(k * 16, 16)
                    terms = [rows[off + rr, sl] * cvecs[rr]
                             for rr in range(16)]
                    while len(terms) > 1:
                        terms = [terms[i] + terms[i + 1]
                                 for i in range(0, len(terms), 2)]
                    acc[l, sl] = acc[l, sl] + terms[0]

            @pl.when(jnp.logical_not(onseg))
            def _slow():
                locc = jnp.clip(loc, 0, _SPW - 1)
                for rr in range(16):
                    r = off + rr
                    l = locc[rr]
                    for k in range(_H // 16):
                        sl = pl.ds(k * 16, 16)
                        acc[l, sl] = acc[l, sl] + rows[r, sl] * cvecs[rr]
            return carry2
        lax.fori_loop(0, _C // 16, group, carry)
        return carry
    lax.fori_loop(0, nchunks, chunk, 0)

    base = pl.multiple_of(b0, _SPW)
    pltpu.sync_copy(acc, out_hbm.at[pl.ds(base, _SPW)])


def _s1_body(node_hbm, batch_hbm, starts_hbm, out_hbm,
             rows, idx_v, st_v, acc):
    _seg_body(False, node_hbm, batch_hbm, starts_hbm, None,
              out_hbm, rows, idx_v, None, st_v, acc)


def _c_body(node_hbm, batch_hbm, starts_hbm, e_hbm, out_hbm,
            rows, idx_v, sc_v, st_v, acc):
    _seg_body(True, node_hbm, batch_hbm, starts_hbm, e_hbm,
              out_hbm, rows, idx_v, sc_v, st_v, acc)


_out_type = jax.ShapeDtypeStruct((_B, _H), jnp.float32)

_s1 = pl.kernel(
    _s1_body, out_type=_out_type, mesh=_mesh,
    scratch_types=[
        pltpu.VMEM((_C, _H), jnp.float32),
        pltpu.VMEM((_C,), jnp.int32),
        pltpu.VMEM((_B + 16,), jnp.int32),
        pltpu.VMEM((_SPW, _H), jnp.float32),
    ])

_c = pl.kernel(
    _c_body, out_type=_out_type, mesh=_mesh,
    scratch_types=[
        pltpu.VMEM((_C, _H), jnp.float32),
        pltpu.VMEM((_C,), jnp.int32),
        pltpu.VMEM((_C,), jnp.float32),
        pltpu.VMEM((_B + 16,), jnp.int32),
        pltpu.VMEM((_SPW, _H), jnp.float32),
    ])


def kernel(node_feats, batch, W_attend, gru_w_ih, gru_w_hh, gru_b_ih,
           gru_b_hh):
    batch3 = batch.reshape(_G, 1, _NB)
    node_spec = pl.BlockSpec((_NB, _H), lambda i: (i, 0))
    batch_spec = pl.BlockSpec((1, 1, _NB), lambda i: (i, 0, 0))
    scores_spec = pl.BlockSpec((1, 1, _NB), lambda i: (i, 0, 0))

    att, counts = pl.pallas_call(
        _k1_body,
        grid=(_G,),
        in_specs=[node_spec, _full((_H, _H)), batch_spec],
        out_specs=[node_spec, _full((_B, 1))],
        out_shape=[
            jax.ShapeDtypeStruct((_N, _H), jnp.float32),
            jax.ShapeDtypeStruct((_B, 1), jnp.float32),
        ],
    )(node_feats, W_attend, batch3)

    starts = pl.pallas_call(
        _kst_body,
        in_specs=[_full((_B, 1))],
        out_specs=_full((_B, 1)),
        out_shape=jax.ShapeDtypeStruct((_B, 1), jnp.int32),
    )(counts)
    starts1 = starts.reshape(_B)
    sums = _s1(node_feats, batch, starts1)

    gf = pl.pallas_call(
        _k2_body,
        in_specs=[_full((_B, _H)), _full((_B, 1))],
        out_specs=_full((_B, _H)),
        out_shape=jax.ShapeDtypeStruct((_B, _H), jnp.float32),
    )(sums, counts)

    for _ in range(_T):
        scores, m = pl.pallas_call(
            _ka_body,
            grid=(_G,),
            in_specs=[node_spec, _full((_B, _H)), batch_spec],
            out_specs=[scores_spec, _full((_B, 1))],
            out_shape=[
                jax.ShapeDtypeStruct((_G, 1, _NB), jnp.float32),
                jax.ShapeDtypeStruct((_B, 1), jnp.float32),
            ],
        )(att, gf, batch3)

        ev, ssum = pl.pallas_call(
            _ke_body,
            grid=(_G,),
            in_specs=[batch_spec, scores_spec, _full((_B, 1))],
            out_specs=[scores_spec, _full((_B, 1))],
            out_shape=[
                jax.ShapeDtypeStruct((_G, 1, _NB), jnp.float32),
                jax.ShapeDtypeStruct((_B, 1), jnp.float32),
            ],
        )(batch3, scores, m)

        csum = _c(node_feats, batch, starts1, ev.reshape(_N))

        gf = pl.pallas_call(
            _kg_body,
            in_specs=[_full((_B, _H)), _full((_B, 1)), _full((_B, _H)),
                      _full((3 * _H, _H)), _full((3 * _H, _H)),
                      _full((3 * _H,)), _full((3 * _H,))],
            out_specs=_full((_B, _H)),
            out_shape=jax.ShapeDtypeStruct((_B, _H), jnp.float32),
        )(csum, ssum, gf, gru_w_ih, gru_w_hh, gru_b_ih, gru_b_hh)

    return gf
